# baseline (speedup 1.0000x reference)
import functools

import jax
import jax.numpy as jnp
from jax import lax
from jax.experimental import pallas as pl
from jax.experimental.pallas import tpu as pltpu
from jax.experimental.pallas import tpu_sc as plsc

TOP_K = 8
N_EXPERTS = 64
N_TOKENS = 16384
ROWS_PER_W = 512


def _keys_kernel(x_ref, wt_ref, b_ref, keys_ref):
    logits = jnp.dot(x_ref[:], wt_ref[:], preferred_element_type=jnp.float32)
    logits = logits + b_ref[:]
    rows = logits.shape[0]
    lane = lax.broadcasted_iota(jnp.int32, (rows, N_EXPERTS), 1)
    e = jnp.exp(logits - jnp.max(logits, axis=-1, keepdims=True))
    scores = e / jnp.sum(e, axis=-1, keepdims=True)
    sb = lax.bitcast_convert_type(scores, jnp.int32)
    keys = (sb & jnp.int32(-64)) | (jnp.int32(N_EXPERTS - 1) - lane)
    keys_ref[:] = keys.T


def _routing_keys(x, weight, bias):
    n_rows, h = x.shape
    wt = weight.T
    b = bias.reshape(1, N_EXPERTS)
    block_rows = 2048
    return pl.pallas_call(
        _keys_kernel,
        grid=(n_rows // block_rows,),
        in_specs=[
            pl.BlockSpec((block_rows, h), lambda i: (i, 0)),
            pl.BlockSpec((h, N_EXPERTS), lambda i: (0, 0)),
            pl.BlockSpec((1, N_EXPERTS), lambda i: (0, 0)),
        ],
        out_specs=pl.BlockSpec((N_EXPERTS, block_rows), lambda i: (0, i)),
        out_shape=jax.ShapeDtypeStruct((N_EXPERTS, n_rows), jnp.int32),
        compiler_params=pltpu.CompilerParams(
            dimension_semantics=("arbitrary",),
        ),
    )(x, wt, b)


def _make_sc_topk(chunk_rows):
  rows_per_w = chunk_rows // 32

  def _sc_topk_kernel(keys_hbm, idx_hbm, w_hbm, keys_v, out_i_v, out_w_v):
    info = plsc.get_sparse_core_info()
    nc = info.num_cores
    wid = lax.axis_index("s") * nc + lax.axis_index("c")
    base = wid * rows_per_w

    pltpu.sync_copy(keys_hbm.at[:, pl.ds(base, rows_per_w)], keys_v)

    def body(g, carry):
        regs = [jnp.full((16,), -1, jnp.int32)] * TOP_K
        for e in range(N_EXPERTS):
            t = keys_v[e, pl.ds(g * 16, 16)]
            for j in range(TOP_K):
                nr = jnp.maximum(regs[j], t)
                t = jnp.minimum(regs[j], t)
                regs[j] = nr
        tvs = []
        for j in range(TOP_K):
            tvs.append(lax.bitcast_convert_type(regs[j] & jnp.int32(-64),
                                                jnp.float32))
        denom = tvs[0]
        for j in range(1, TOP_K):
            denom = denom + tvs[j]
        denom = denom + 1e-20
        for j in range(TOP_K):
            ti = jnp.int32(N_EXPERTS - 1) - (regs[j] & jnp.int32(N_EXPERTS - 1))
            out_i_v[j, pl.ds(g * 16, 16)] = ti
            out_w_v[j, pl.ds(g * 16, 16)] = tvs[j] / denom
        return carry

    lax.fori_loop(0, rows_per_w // 16, body, 0)

    pltpu.sync_copy(out_i_v, idx_hbm.at[:, pl.ds(base, rows_per_w)])
    pltpu.sync_copy(out_w_v, w_hbm.at[:, pl.ds(base, rows_per_w)])

  return functools.partial(
      pl.kernel,
      mesh=plsc.VectorSubcoreMesh(core_axis_name="c", subcore_axis_name="s"),
      out_type=[
          jax.ShapeDtypeStruct((TOP_K, chunk_rows), jnp.int32),
          jax.ShapeDtypeStruct((TOP_K, chunk_rows), jnp.float32),
      ],
      scratch_types=[
          pltpu.VMEM((N_EXPERTS, rows_per_w), jnp.int32),
          pltpu.VMEM((TOP_K, rows_per_w), jnp.int32),
          pltpu.VMEM((TOP_K, rows_per_w), jnp.float32),
      ],
  )(_sc_topk_kernel)


@functools.partial(jax.jit, static_argnames=())
def kernel(x, weight, bias):
    bsz, seq_len, h = x.shape
    n_rows = bsz * seq_len
    xf = x.reshape(n_rows, h)
    n_chunks = 2
    chunk = n_rows // n_chunks
    sc_topk = _make_sc_topk(chunk)
    parts = []
    for i in range(n_chunks):
        keys = _routing_keys(xf[i * chunk:(i + 1) * chunk], weight, bias)
        parts.append(sc_topk(keys))
    idx_t = jnp.concatenate([p[0] for p in parts], axis=1)
    w_t = jnp.concatenate([p[1] for p in parts], axis=1)
    aux_loss = jnp.asarray(0.0, dtype=jnp.float32)
    return (idx_t.T, w_t.T, aux_loss)


# speedup vs baseline: 2.1644x; 2.1644x over previous
import functools

import jax
import jax.numpy as jnp
from jax import lax
from jax.experimental import pallas as pl
from jax.experimental.pallas import tpu as pltpu
from jax.experimental.pallas import tpu_sc as plsc

TOP_K = 8
N_EXPERTS = 64
N_TOKENS = 16384
ROWS_PER_W = 512


def _keys_kernel(x_ref, w_ref, b_ref, keys_ref):
    logits = lax.dot_general(w_ref[:], x_ref[:], (((1,), (1,)), ((), ())),
                             preferred_element_type=jnp.float32)
    logits = logits + b_ref[:]
    rows = logits.shape[1]
    expert = lax.broadcasted_iota(jnp.int32, (N_EXPERTS, rows), 0)
    e = jnp.exp(logits - jnp.max(logits, axis=0, keepdims=True))
    scores = e / jnp.sum(e, axis=0, keepdims=True)
    sb = lax.bitcast_convert_type(scores, jnp.int32)
    keys_ref[:] = (sb & jnp.int32(-64)) | (jnp.int32(N_EXPERTS - 1) - expert)


def _routing_keys(x, weight, bias):
    n_rows, h = x.shape
    b = bias.reshape(N_EXPERTS, 1)
    block_rows = 2048
    return pl.pallas_call(
        _keys_kernel,
        grid=(n_rows // block_rows,),
        in_specs=[
            pl.BlockSpec((block_rows, h), lambda i: (i, 0)),
            pl.BlockSpec((N_EXPERTS, h), lambda i: (0, 0)),
            pl.BlockSpec((N_EXPERTS, 1), lambda i: (0, 0)),
        ],
        out_specs=pl.BlockSpec((N_EXPERTS, block_rows), lambda i: (0, i)),
        out_shape=jax.ShapeDtypeStruct((N_EXPERTS, n_rows), jnp.int32),
        compiler_params=pltpu.CompilerParams(
            dimension_semantics=("arbitrary",),
        ),
    )(x, weight, b)


def _sc_topk_kernel(keys_hbm, idx_hbm, w_hbm, keys_v, out_i_v, out_w_v):
    info = plsc.get_sparse_core_info()
    nc = info.num_cores
    wid = lax.axis_index("s") * nc + lax.axis_index("c")
    base = wid * ROWS_PER_W

    pltpu.sync_copy(keys_hbm.at[:, pl.ds(base, ROWS_PER_W)], keys_v)

    def body(g, carry):
        regs = [jnp.full((16,), -1, jnp.int32)] * TOP_K
        for e in range(N_EXPERTS):
            t = keys_v[e, pl.ds(g * 16, 16)]
            for j in range(TOP_K):
                nr = jnp.maximum(regs[j], t)
                t = jnp.minimum(regs[j], t)
                regs[j] = nr
        tvs = []
        for j in range(TOP_K):
            tvs.append(lax.bitcast_convert_type(regs[j] & jnp.int32(-64),
                                                jnp.float32))
        denom = tvs[0]
        for j in range(1, TOP_K):
            denom = denom + tvs[j]
        denom = denom + 1e-20
        for j in range(TOP_K):
            ti = jnp.int32(N_EXPERTS - 1) - (regs[j] & jnp.int32(N_EXPERTS - 1))
            out_i_v[j, pl.ds(g * 16, 16)] = ti
            out_w_v[j, pl.ds(g * 16, 16)] = tvs[j] / denom
        return carry

    lax.fori_loop(0, ROWS_PER_W // 16, body, 0)

    pltpu.sync_copy(out_i_v, idx_hbm.at[:, pl.ds(base, ROWS_PER_W)])
    pltpu.sync_copy(out_w_v, w_hbm.at[:, pl.ds(base, ROWS_PER_W)])


_sc_topk = functools.partial(
    pl.kernel,
    mesh=plsc.VectorSubcoreMesh(core_axis_name="c", subcore_axis_name="s"),
    out_type=[
        jax.ShapeDtypeStruct((TOP_K, N_TOKENS), jnp.int32),
        jax.ShapeDtypeStruct((TOP_K, N_TOKENS), jnp.float32),
    ],
    scratch_types=[
        pltpu.VMEM((N_EXPERTS, ROWS_PER_W), jnp.int32),
        pltpu.VMEM((TOP_K, ROWS_PER_W), jnp.int32),
        pltpu.VMEM((TOP_K, ROWS_PER_W), jnp.float32),
    ],
)(_sc_topk_kernel)


@functools.partial(jax.jit, static_argnames=())
def kernel(x, weight, bias):
    bsz, seq_len, h = x.shape
    n_rows = bsz * seq_len
    xf = x.reshape(n_rows, h)
    keys = _routing_keys(xf, weight, bias)
    idx_t, w_t = _sc_topk(keys)
    aux_loss = jnp.asarray(0.0, dtype=jnp.float32)
    return (idx_t.T, w_t.T, aux_loss)


# speedup vs baseline: 2.2677x; 1.0477x over previous
import functools

import jax
import jax.numpy as jnp
from jax import lax
from jax.experimental import pallas as pl
from jax.experimental.pallas import tpu as pltpu
from jax.experimental.pallas import tpu_sc as plsc

TOP_K = 8
N_EXPERTS = 64
N_TOKENS = 16384
ROWS_PER_W = 512

_SORT8 = [(0, 1), (2, 3), (4, 5), (6, 7),
          (0, 2), (1, 3), (4, 6), (5, 7),
          (1, 2), (5, 6),
          (0, 4), (1, 5), (2, 6), (3, 7),
          (2, 4), (3, 5),
          (1, 2), (3, 4), (5, 6)]
_BITONIC8 = [(0, 4), (1, 5), (2, 6), (3, 7),
             (0, 2), (1, 3), (4, 6), (5, 7),
             (0, 1), (2, 3), (4, 5), (6, 7)]


def _keys_kernel(x_ref, w_ref, b_ref, keys_ref):
    logits = lax.dot_general(w_ref[:], x_ref[:], (((1,), (1,)), ((), ())),
                             preferred_element_type=jnp.float32)
    logits = logits + b_ref[:]
    rows = logits.shape[1]
    expert = lax.broadcasted_iota(jnp.int32, (N_EXPERTS, rows), 0)
    e = jnp.exp(logits - jnp.max(logits, axis=0, keepdims=True))
    scores = e / jnp.sum(e, axis=0, keepdims=True)
    sb = lax.bitcast_convert_type(scores, jnp.int32)
    keys_ref[:] = (sb & jnp.int32(-64)) | (jnp.int32(N_EXPERTS - 1) - expert)


def _routing_keys(x, weight, bias):
    n_rows, h = x.shape
    b = bias.reshape(N_EXPERTS, 1)
    block_rows = 2048
    return pl.pallas_call(
        _keys_kernel,
        grid=(n_rows // block_rows,),
        in_specs=[
            pl.BlockSpec((block_rows, h), lambda i: (i, 0)),
            pl.BlockSpec((N_EXPERTS, h), lambda i: (0, 0)),
            pl.BlockSpec((N_EXPERTS, 1), lambda i: (0, 0)),
        ],
        out_specs=pl.BlockSpec((N_EXPERTS, block_rows), lambda i: (0, i)),
        out_shape=jax.ShapeDtypeStruct((N_EXPERTS, n_rows), jnp.int32),
        compiler_params=pltpu.CompilerParams(
            dimension_semantics=("arbitrary",),
        ),
    )(x, weight, b)


def _sc_topk_kernel(keys_hbm, idx_hbm, w_hbm, keys_v, out_i_v, out_w_v):
    info = plsc.get_sparse_core_info()
    nc = info.num_cores
    wid = lax.axis_index("s") * nc + lax.axis_index("c")
    base = wid * ROWS_PER_W

    pltpu.sync_copy(keys_hbm.at[:, pl.ds(base, ROWS_PER_W)], keys_v)

    def body(g, carry):
        regs = None
        for c in range(N_EXPERTS // TOP_K):
            chunk = [keys_v[c * TOP_K + e, pl.ds(g * 16, 16)]
                     for e in range(TOP_K)]
            for i, j in _SORT8:
                hi = jnp.maximum(chunk[i], chunk[j])
                lo = jnp.minimum(chunk[i], chunk[j])
                chunk[i], chunk[j] = hi, lo
            if regs is None:
                regs = chunk
                continue
            merged = [jnp.maximum(regs[i], chunk[TOP_K - 1 - i])
                      for i in range(TOP_K)]
            for i, j in _BITONIC8:
                hi = jnp.maximum(merged[i], merged[j])
                lo = jnp.minimum(merged[i], merged[j])
                merged[i], merged[j] = hi, lo
            regs = merged
        tvs = []
        for j in range(TOP_K):
            tvs.append(lax.bitcast_convert_type(regs[j] & jnp.int32(-64),
                                                jnp.float32))
        denom = tvs[0]
        for j in range(1, TOP_K):
            denom = denom + tvs[j]
        denom = denom + 1e-20
        for j in range(TOP_K):
            ti = jnp.int32(N_EXPERTS - 1) - (regs[j] & jnp.int32(N_EXPERTS - 1))
            out_i_v[j, pl.ds(g * 16, 16)] = ti
            out_w_v[j, pl.ds(g * 16, 16)] = tvs[j] / denom
        return carry

    lax.fori_loop(0, ROWS_PER_W // 16, body, 0)

    pltpu.sync_copy(out_i_v, idx_hbm.at[:, pl.ds(base, ROWS_PER_W)])
    pltpu.sync_copy(out_w_v, w_hbm.at[:, pl.ds(base, ROWS_PER_W)])


_sc_topk = functools.partial(
    pl.kernel,
    mesh=plsc.VectorSubcoreMesh(core_axis_name="c", subcore_axis_name="s"),
    out_type=[
        jax.ShapeDtypeStruct((TOP_K, N_TOKENS), jnp.int32),
        jax.ShapeDtypeStruct((TOP_K, N_TOKENS), jnp.float32),
    ],
    scratch_types=[
        pltpu.VMEM((N_EXPERTS, ROWS_PER_W), jnp.int32),
        pltpu.VMEM((TOP_K, ROWS_PER_W), jnp.int32),
        pltpu.VMEM((TOP_K, ROWS_PER_W), jnp.float32),
    ],
)(_sc_topk_kernel)


@functools.partial(jax.jit, static_argnames=())
def kernel(x, weight, bias):
    bsz, seq_len, h = x.shape
    n_rows = bsz * seq_len
    xf = x.reshape(n_rows, h)
    keys = _routing_keys(xf, weight, bias)
    idx_t, w_t = _sc_topk(keys)
    aux_loss = jnp.asarray(0.0, dtype=jnp.float32)
    return (idx_t.T, w_t.T, aux_loss)
